# baseline (speedup 1.0000x reference)
import functools

import jax
import jax.numpy as jnp
from jax import lax
from jax.experimental import pallas as pl
from jax.experimental.pallas import tpu as pltpu
from jax.experimental.pallas import tpu_sc as plsc

N = 50000
E = 800000
D = 64
RB = 512
P = 50176
NBLK = P // RB
SR = P // 128
HALF = 25000
PH = 25008
TROWS = PH // 16
CH = 128
EROWS = E // CH
KS = [40000, 32000, 25600]
HI = jax.lax.Precision.HIGHEST


def _mm_body(x_ref, wn_ref, wr_ref, h_ref, xr_ref):
    xb = x_ref[...]
    h_ref[...] = jnp.dot(xb, wn_ref[...], precision=HI,
                         preferred_element_type=jnp.float32)
    xr_ref[...] = jnp.dot(xb, wr_ref[...], precision=HI,
                          preferred_element_type=jnp.float32)


def _mm(x, wn, wr):
    return pl.pallas_call(
        _mm_body,
        grid=(NBLK,),
        in_specs=[
            pl.BlockSpec((RB, D), lambda i: (i, 0)),
            pl.BlockSpec((D, D), lambda i: (0, 0)),
            pl.BlockSpec((D, D), lambda i: (0, 0)),
        ],
        out_specs=[
            pl.BlockSpec((RB, D), lambda i: (i, 0)),
            pl.BlockSpec((RB, D), lambda i: (i, 0)),
        ],
        out_shape=[
            jax.ShapeDtypeStruct((P, D), jnp.float32),
            jax.ShapeDtypeStruct((P, D), jnp.float32),
        ],
    )(x, wn, wr)


def _sc_body(h_hbm, ei0_hbm, ei1_hbm, zin_hbm, agg_hbm,
             src2d, dst2d, rows, aggsp,
             sg0, sg1, sg2, ss0, ss1, ss2):
    c = lax.axis_index("c")
    s = lax.axis_index("s")
    base_c = c * HALF
    for r in range(TROWS // CH):
        pltpu.sync_copy(zin_hbm, aggsp.at[pl.ds(s * TROWS + r * CH, CH)])
    pltpu.sync_copy(zin_hbm.at[pl.ds(0, TROWS % CH)],
                    aggsp.at[pl.ds(s * TROWS + (TROWS // CH) * CH, TROWS % CH)])
    plsc.subcore_barrier()

    base_row = s * 390 + jnp.minimum(s, 10)
    nrows = jnp.where(s < 10, 391, 390)
    sems_g = (sg0, sg1, sg2)
    sems_s = (ss0, ss1, ss2)

    def stage_a(b, grow):
        pltpu.sync_copy(ei0_hbm.at[pl.ds(grow, 1)], src2d.at[pl.ds(b, 1)])
        pltpu.sync_copy(ei1_hbm.at[pl.ds(grow, 1)], dst2d.at[pl.ds(b, 1)])
        for j in range(8):
            d = dst2d[b, pl.ds(j * 16, 16)]
            inr = (d >= base_c) & (d < base_c + HALF)
            dst2d[b, pl.ds(j * 16, 16)] = jnp.where(
                inr, d - base_c, HALF + (d & 7))
        pltpu.async_copy(h_hbm.at[src2d.at[b]], rows.at[b], sems_g[b])

    def drain_g(b):
        pltpu.make_async_copy(h_hbm.at[src2d.at[b]], rows.at[b],
                              sems_g[b]).wait()

    def fire_s(b):
        pltpu.async_copy(rows.at[b], aggsp.at[dst2d.at[b]], sems_s[b],
                         add=True)

    def drain_s(b):
        pltpu.make_async_copy(rows.at[b], aggsp.at[dst2d.at[b]],
                              sems_s[b]).wait()

    stage_a(0, base_row)
    stage_a(1, base_row + 1)
    stage_a(2, base_row + 2)

    def body(i, carry):
        for b in range(3):
            drain_g(b)
            fire_s(b)
        for b in range(3):
            drain_s(b)
            stage_a(b, base_row + 3 * i + 3 + b)
        return carry

    lax.fori_loop(0, 129, body, 0)
    for b in range(3):
        drain_g(b)
        fire_s(b)
    for b in range(3):
        drain_s(b)

    def tbody(i, carry):
        row = base_row + 390 + i
        pltpu.sync_copy(ei0_hbm.at[pl.ds(row, 1)], src2d.at[pl.ds(0, 1)])
        pltpu.sync_copy(ei1_hbm.at[pl.ds(row, 1)], dst2d.at[pl.ds(0, 1)])
        for j in range(8):
            d = dst2d[0, pl.ds(j * 16, 16)]
            inr = (d >= base_c) & (d < base_c + HALF)
            dst2d[0, pl.ds(j * 16, 16)] = jnp.where(
                inr, d - base_c, HALF + (d & 7))
        pltpu.async_copy(h_hbm.at[src2d.at[0]], rows.at[0], sg0).wait()
        pltpu.sync_copy(rows.at[0], aggsp.at[dst2d.at[0]], add=True)
        return carry

    lax.fori_loop(0, nrows - 390, tbody, 0)
    plsc.subcore_barrier()

    def wb(loc, n):
        pltpu.sync_copy(aggsp.at[pl.ds(loc, n)], agg_hbm.at[pl.ds(base_c + loc, n)])

    @pl.when(s < 15)
    def _():
        for r in range(TROWS // CH):
            wb(s * TROWS + r * CH, CH)
        wb(s * TROWS + (TROWS // CH) * CH, TROWS % CH)

    @pl.when(s == 15)
    def _():
        last = HALF - 15 * TROWS
        for r in range(last // CH):
            wb(15 * TROWS + r * CH, CH)
        wb(15 * TROWS + (last // CH) * CH, last % CH)

    @pl.when((c == 1) & (s == 15))
    def _():
        pltpu.sync_copy(zin_hbm, rows.at[0])
        pltpu.sync_copy(rows.at[0], agg_hbm.at[pl.ds(N, CH)])
        pltpu.sync_copy(rows.at[0, pl.ds(0, P - N - CH)],
                        agg_hbm.at[pl.ds(N + CH, P - N - CH)])


_sc_call = functools.partial(
    pl.kernel,
    mesh=plsc.VectorSubcoreMesh(core_axis_name="c", subcore_axis_name="s"),
    compiler_params=pltpu.CompilerParams(use_tc_tiling_on_sc=False),
    out_type=jax.ShapeDtypeStruct((P, D), jnp.float32),
    scratch_types=[
        pltpu.VMEM((3, CH), jnp.int32),
        pltpu.VMEM((3, CH), jnp.int32),
        pltpu.VMEM((3, CH, D), jnp.float32),
        pltpu.VMEM_SHARED((PH, D), jnp.float32),
        pltpu.SemaphoreType.DMA,
        pltpu.SemaphoreType.DMA,
        pltpu.SemaphoreType.DMA,
        pltpu.SemaphoreType.DMA,
        pltpu.SemaphoreType.DMA,
        pltpu.SemaphoreType.DMA,
    ],
)(_sc_body)


def _k1_body(xr_ref, agg_ref, b_ref, p_ref, kept_ref, out_ref, sraw_ref):
    pr = p_ref[...]
    nrm = jnp.sqrt(jnp.sum(pr * pr))
    o = jnp.maximum(xr_ref[...] + agg_ref[...] + b_ref[...], 0.0)
    o = jnp.where(kept_ref[...] > 0, o, 0.0)
    out_ref[...] = o
    sraw_ref[...] = lax.dot_general(o, pr / nrm, (((1,), (1,)), ((), ())),
                                    precision=HI,
                                    preferred_element_type=jnp.float32)


def _k1(xr, agg, b, p, kept):
    return pl.pallas_call(
        _k1_body,
        grid=(NBLK,),
        in_specs=[
            pl.BlockSpec((RB, D), lambda i: (i, 0)),
            pl.BlockSpec((RB, D), lambda i: (i, 0)),
            pl.BlockSpec((1, D), lambda i: (0, 0)),
            pl.BlockSpec((1, D), lambda i: (0, 0)),
            pl.BlockSpec((RB, 1), lambda i: (i, 0)),
        ],
        out_specs=[
            pl.BlockSpec((RB, D), lambda i: (i, 0)),
            pl.BlockSpec((RB, 1), lambda i: (i, 0)),
        ],
        out_shape=[
            jax.ShapeDtypeStruct((P, D), jnp.float32),
            jax.ShapeDtypeStruct((P, 1), jnp.float32),
        ],
    )(xr, agg, b, p, kept)


def _sel_body(k, sraw_ref, kept_ref, keptn_ref, gate_ref):
    score = jnp.tanh(sraw_ref[...])
    sc = jnp.where(kept_ref[...] > 0, score, -3.0)
    u = lax.bitcast_convert_type(sc, jnp.int32)
    ukey = jnp.where(u < 0, ~u, u ^ jnp.int32(-2147483648)).astype(jnp.uint32)

    def vbody(t, pacc):
        cand = pacc | (jnp.uint32(1) << (jnp.uint32(31) - t.astype(jnp.uint32)))
        cnt = jnp.sum((ukey >= cand).astype(jnp.int32))
        return jnp.where(cnt >= k, cand, pacc)

    v = lax.fori_loop(0, 32, vbody, jnp.uint32(0))
    g = jnp.sum((ukey > v).astype(jnp.int32))
    need = k - g
    eq = ukey == v
    idx = (lax.broadcasted_iota(jnp.int32, (SR, 128), 0) * 128
           + lax.broadcasted_iota(jnp.int32, (SR, 128), 1))

    def jbody(t, acc):
        cand = acc | (jnp.int32(1) << (jnp.int32(16) - t))
        cnt = jnp.sum((eq & (idx < cand)).astype(jnp.int32))
        return jnp.where(cnt < need, cand, acc)

    jcut = lax.fori_loop(0, 17, jbody, jnp.int32(0))
    keep = (ukey > v) | (eq & (idx <= jcut))
    keptn_ref[...] = keep.astype(jnp.float32)
    gate_ref[...] = jnp.where(keep, score, 0.0)


def _select(k, sraw2, kept2):
    return pl.pallas_call(
        functools.partial(_sel_body, k),
        in_specs=[
            pl.BlockSpec((SR, 128), lambda: (0, 0)),
            pl.BlockSpec((SR, 128), lambda: (0, 0)),
        ],
        out_specs=[
            pl.BlockSpec((SR, 128), lambda: (0, 0)),
            pl.BlockSpec((SR, 128), lambda: (0, 0)),
        ],
        out_shape=[
            jax.ShapeDtypeStruct((SR, 128), jnp.float32),
            jax.ShapeDtypeStruct((SR, 128), jnp.float32),
        ],
    )(sraw2, kept2)


def _k2_body(kdiv, last, out_ref, gate_ref, kept_ref, wn_ref, wr_ref, *refs):
    if last:
        pool_ref, acc_ref = refs
    else:
        h_ref, xr_ref, pool_ref, acc_ref = refs
    i = pl.program_id(0)

    @pl.when(i == 0)
    def _():
        acc_ref[0:1, :] = jnp.full((1, D), -3e38, jnp.float32)
        acc_ref[1:2, :] = jnp.zeros((1, D), jnp.float32)

    xn = out_ref[...] * gate_ref[...]
    mx = jnp.max(jnp.where(kept_ref[...] > 0, xn, -3e38), axis=0, keepdims=True)
    sm = jnp.sum(xn, axis=0, keepdims=True)
    acc_ref[0:1, :] = jnp.maximum(acc_ref[0:1, :], mx)
    acc_ref[1:2, :] = acc_ref[1:2, :] + sm
    if not last:
        h_ref[...] = jnp.dot(xn, wn_ref[...], precision=HI,
                             preferred_element_type=jnp.float32)
        xr_ref[...] = jnp.dot(xn, wr_ref[...], precision=HI,
                              preferred_element_type=jnp.float32)

    @pl.when(i == NBLK - 1)
    def _():
        pool_ref[0:1, :] = acc_ref[0:1, :]
        pool_ref[1:2, :] = acc_ref[1:2, :] * (1.0 / kdiv)


def _k2(out, gate, kept, wn, wr, kdiv, last):
    out_specs = [] if last else [
        pl.BlockSpec((RB, D), lambda i: (i, 0)),
        pl.BlockSpec((RB, D), lambda i: (i, 0)),
    ]
    out_shape = [] if last else [
        jax.ShapeDtypeStruct((P, D), jnp.float32),
        jax.ShapeDtypeStruct((P, D), jnp.float32),
    ]
    res = pl.pallas_call(
        functools.partial(_k2_body, kdiv, last),
        grid=(NBLK,),
        in_specs=[
            pl.BlockSpec((RB, D), lambda i: (i, 0)),
            pl.BlockSpec((RB, 1), lambda i: (i, 0)),
            pl.BlockSpec((RB, 1), lambda i: (i, 0)),
            pl.BlockSpec((D, D), lambda i: (0, 0)),
            pl.BlockSpec((D, D), lambda i: (0, 0)),
        ],
        out_specs=out_specs + [pl.BlockSpec((2, D), lambda i: (0, 0))],
        out_shape=out_shape + [jax.ShapeDtypeStruct((2, D), jnp.float32)],
        scratch_shapes=[pltpu.VMEM((8, D), jnp.float32)],
    )(out, gate, kept, wn, wr)
    return res[0] if last else res


def _head_body(p0_ref, p1_ref, p2_ref, w1_ref, b1_ref, w2_ref, b2_ref, o_ref):
    sm = p0_ref[...] + p1_ref[...] + p2_ref[...]
    cat = jnp.concatenate([sm[0:1, :], sm[1:2, :]], axis=1)
    h1 = jnp.maximum(jnp.dot(cat, w1_ref[...], precision=HI,
                             preferred_element_type=jnp.float32) + b1_ref[...], 0.0)
    o_ref[...] = jnp.dot(h1, w2_ref[...], precision=HI,
                         preferred_element_type=jnp.float32) + b2_ref[...]


def _head(p0, p1, p2, w1, b1, w2, b2):
    return pl.pallas_call(
        _head_body,
        out_shape=jax.ShapeDtypeStruct((1, 128), jnp.float32),
    )(p0, p1, p2, w1, b1, w2, b2)


def kernel(x, edge_index, batch, W_root0, W_nbr0, b0, p0, W_root1, W_nbr1, b1,
           p1, W_root2, W_nbr2, b2, p2, lin1_w, lin1_b, lin2_w, lin2_b):
    del batch
    xpad = jnp.pad(x, ((0, P - N), (0, 0)))
    ei0 = edge_index[0].reshape(EROWS, CH)
    ei1 = edge_index[1].reshape(EROWS, CH)
    zin = jnp.zeros((CH, D), jnp.float32)
    kept = (jnp.arange(P, dtype=jnp.int32) < N).astype(jnp.float32)[:, None]
    Ws = [(W_root0, W_nbr0, b0, p0), (W_root1, W_nbr1, b1, p1),
          (W_root2, W_nbr2, b2, p2)]
    h, xr = _mm(xpad, W_nbr0, W_root0)
    pools = []
    for n in range(3):
        _, _, b, p = Ws[n]
        agg = _sc_call(h, ei0, ei1, zin)
        out, sraw = _k1(xr, agg, b.reshape(1, D), p.reshape(1, D), kept)
        keptn2, gate2 = _select(KS[n], sraw.reshape(SR, 128), kept.reshape(SR, 128))
        kept = keptn2.reshape(P, 1)
        gate = gate2.reshape(P, 1)
        if n < 2:
            wr_n, wn_n = Ws[n + 1][0], Ws[n + 1][1]
            h, xr, pool = _k2(out, gate, kept, wn_n, wr_n, KS[n], False)
        else:
            dummy_w = jnp.zeros((D, D), jnp.float32)
            pool = _k2(out, gate, kept, dummy_w, dummy_w, KS[n], True)
        pools.append(pool)
    o = _head(pools[0], pools[1], pools[2], lin1_w, lin1_b.reshape(1, D),
              jnp.pad(lin2_w, ((0, 0), (0, 122))),
              jnp.pad(lin2_b, (0, 122)).reshape(1, 128))
    return o[:, :6]
